# Initial kernel scaffold; baseline (speedup 1.0000x reference)
import jax
import jax.numpy as jnp
from jax.experimental import pallas as pl


def kernel(x, batch):
    raise NotImplementedError("write your pallas kernel here")


# speedup vs baseline: 4.0383x; 4.0383x over previous
import functools

import jax
import jax.numpy as jnp
from jax import lax
from jax.experimental import pallas as pl
from jax.experimental.pallas import tpu as pltpu
from jax.experimental.pallas import tpu_sc as plsc

_N = 100000
_G = 128
_ROWS = 1000


def _rsqrt_newton(x):
    i = plsc.bitcast(x, jnp.int32)
    i = jnp.int32(0x5F3759DF) - lax.shift_right_arithmetic(i, 1)
    y = plsc.bitcast(i, jnp.float32)
    for _ in range(3):
        y = y * (1.5 - 0.5 * x * y * y)
    return y


def _sc_inv_sqrt_deg(batch):
    mesh = plsc.VectorSubcoreMesh(core_axis_name="c", subcore_axis_name="s")

    @functools.partial(
        pl.kernel,
        mesh=mesh,
        compiler_params=pltpu.CompilerParams(needs_layout_passes=False),
        out_type=jax.ShapeDtypeStruct((_G,), jnp.float32),
        scratch_types=[
            pltpu.VMEM((_N,), jnp.int32),
            pltpu.VMEM((_G + 16,), jnp.int32),
            pltpu.VMEM((_G,), jnp.float32),
        ],
    )
    def k(batch_hbm, out_hbm, b_v, ub_v, inv_v):
        wid = lax.axis_index("s") * 2 + lax.axis_index("c")

        @pl.when(wid == 0)
        def _():
            pltpu.sync_copy(batch_hbm, b_v)
            lane = lax.broadcasted_iota(jnp.int32, (16,), 0)
            zeros = jnp.zeros((16,), jnp.int32)
            ub_v[pl.ds(0, 16)] = zeros
            for k8 in range(_G // 16):
                g = lane + (16 * k8)
                lo = zeros
                hi = jnp.full((16,), _N, jnp.int32)
                for _ in range(17):
                    active = lo < hi
                    mid = lax.shift_right_arithmetic(lo + hi, 1)
                    v = plsc.load_gather(b_v, [jnp.minimum(mid, _N - 1)])
                    take = jnp.logical_and(active, v <= g)
                    lo = jnp.where(take, mid + 1, lo)
                    hi = jnp.where(jnp.logical_and(active, v > g), mid, hi)
                ub_v[pl.ds(8 + 16 * k8, 16)] = lo
            for k8 in range(_G // 16):
                cur = ub_v[pl.ds(8 + 16 * k8, 16)]
                prev = plsc.load_gather(ub_v, [lane + (7 + 16 * k8)])
                deg = (cur - prev).astype(jnp.float32)
                inv_v[pl.ds(16 * k8, 16)] = _rsqrt_newton(deg)
            pltpu.sync_copy(inv_v, out_hbm)

    return k(batch)


def _scale_body(x_ref, b_ref, inv_ref, o_ref):
    b = b_ref[0, 0, :]
    inv = inv_ref[0, :]
    gid = lax.broadcasted_iota(jnp.int32, (_ROWS, _G), 1)
    eq = b[:, None] == gid
    scale = jnp.sum(jnp.where(eq, inv[None, :], 0.0), axis=1)
    o_ref[:, :] = x_ref[:, :] * scale[:, None]


def kernel(x, batch):
    b32 = batch.astype(jnp.int32)
    inv = _sc_inv_sqrt_deg(b32)
    batch3 = b32.reshape(_N // _ROWS, 1, _ROWS)
    return pl.pallas_call(
        _scale_body,
        grid=(_N // _ROWS,),
        in_specs=[
            pl.BlockSpec((_ROWS, 512), lambda i: (i, 0)),
            pl.BlockSpec((1, 1, _ROWS), lambda i: (i, 0, 0)),
            pl.BlockSpec((1, _G), lambda i: (0, 0)),
        ],
        out_specs=pl.BlockSpec((_ROWS, 512), lambda i: (i, 0)),
        out_shape=jax.ShapeDtypeStruct((_N, 512), jnp.float32),
        compiler_params=pltpu.CompilerParams(
            dimension_semantics=("arbitrary",),
        ),
    )(x, batch3, inv.reshape(1, _G))


# speedup vs baseline: 4.4996x; 1.1142x over previous
import functools

import jax
import jax.numpy as jnp
from jax import lax
from jax.experimental import pallas as pl
from jax.experimental.pallas import tpu as pltpu
from jax.experimental.pallas import tpu_sc as plsc

_N = 100000
_G = 128
_ROWS = 2000


def _rsqrt_newton(x):
    i = plsc.bitcast(x, jnp.int32)
    i = jnp.int32(0x5F3759DF) - lax.shift_right_arithmetic(i, 1)
    y = plsc.bitcast(i, jnp.float32)
    for _ in range(3):
        y = y * (1.5 - 0.5 * x * y * y)
    return y


def _sc_inv_sqrt_deg(batch):
    mesh = plsc.VectorSubcoreMesh(core_axis_name="c", subcore_axis_name="s")

    @functools.partial(
        pl.kernel,
        mesh=mesh,
        compiler_params=pltpu.CompilerParams(needs_layout_passes=False),
        out_type=jax.ShapeDtypeStruct((_G,), jnp.float32),
        scratch_types=[
            pltpu.VMEM((_N,), jnp.int32),
            pltpu.VMEM((_G + 16,), jnp.int32),
            pltpu.VMEM((_G,), jnp.float32),
        ],
    )
    def k(batch_hbm, out_hbm, b_v, ub_v, inv_v):
        wid = lax.axis_index("s") * 2 + lax.axis_index("c")

        @pl.when(wid == 0)
        def _():
            pltpu.sync_copy(batch_hbm, b_v)
            lane = lax.broadcasted_iota(jnp.int32, (16,), 0)
            zeros = jnp.zeros((16,), jnp.int32)
            ub_v[pl.ds(0, 16)] = zeros
            for k8 in range(_G // 16):
                g = lane + (16 * k8)
                lo = zeros
                hi = jnp.full((16,), _N, jnp.int32)
                for _ in range(17):
                    active = lo < hi
                    mid = lax.shift_right_arithmetic(lo + hi, 1)
                    v = plsc.load_gather(b_v, [jnp.minimum(mid, _N - 1)])
                    take = jnp.logical_and(active, v <= g)
                    lo = jnp.where(take, mid + 1, lo)
                    hi = jnp.where(jnp.logical_and(active, v > g), mid, hi)
                ub_v[pl.ds(8 + 16 * k8, 16)] = lo
            for k8 in range(_G // 16):
                cur = ub_v[pl.ds(8 + 16 * k8, 16)]
                prev = plsc.load_gather(ub_v, [lane + (7 + 16 * k8)])
                deg = (cur - prev).astype(jnp.float32)
                inv_v[pl.ds(16 * k8, 16)] = _rsqrt_newton(deg)
            pltpu.sync_copy(inv_v, out_hbm)

    return k(batch)


def _scale_body(x_ref, b_ref, inv_ref, o_ref):
    b = b_ref[0, 0, :]
    inv = inv_ref[0, :]
    gid = lax.broadcasted_iota(jnp.int32, (_ROWS, _G), 1)
    eq = b[:, None] == gid
    scale = jnp.sum(jnp.where(eq, inv[None, :], 0.0), axis=1)
    o_ref[:, :] = x_ref[:, :] * scale[:, None]


def kernel(x, batch):
    b32 = batch.astype(jnp.int32)
    inv = _sc_inv_sqrt_deg(b32)
    batch3 = b32.reshape(_N // _ROWS, 1, _ROWS)
    return pl.pallas_call(
        _scale_body,
        grid=(_N // _ROWS,),
        in_specs=[
            pl.BlockSpec((_ROWS, 512), lambda i: (i, 0)),
            pl.BlockSpec((1, 1, _ROWS), lambda i: (i, 0, 0)),
            pl.BlockSpec((1, _G), lambda i: (0, 0)),
        ],
        out_specs=pl.BlockSpec((_ROWS, 512), lambda i: (i, 0)),
        out_shape=jax.ShapeDtypeStruct((_N, 512), jnp.float32),
        compiler_params=pltpu.CompilerParams(
            dimension_semantics=("arbitrary",),
        ),
    )(x, batch3, inv.reshape(1, _G))


# speedup vs baseline: 4.5698x; 1.0156x over previous
import functools

import jax
import jax.numpy as jnp
from jax import lax
from jax.experimental import pallas as pl
from jax.experimental.pallas import tpu as pltpu
from jax.experimental.pallas import tpu_sc as plsc

_N = 100000
_G = 128
_ROWS = 4000


def _rsqrt_newton(x):
    i = plsc.bitcast(x, jnp.int32)
    i = jnp.int32(0x5F3759DF) - lax.shift_right_arithmetic(i, 1)
    y = plsc.bitcast(i, jnp.float32)
    for _ in range(3):
        y = y * (1.5 - 0.5 * x * y * y)
    return y


def _sc_inv_sqrt_deg(batch):
    mesh = plsc.VectorSubcoreMesh(core_axis_name="c", subcore_axis_name="s")

    @functools.partial(
        pl.kernel,
        mesh=mesh,
        compiler_params=pltpu.CompilerParams(needs_layout_passes=False),
        out_type=jax.ShapeDtypeStruct((_G,), jnp.float32),
        scratch_types=[
            pltpu.VMEM((_N,), jnp.int32),
            pltpu.VMEM((_G + 16,), jnp.int32),
            pltpu.VMEM((_G,), jnp.float32),
        ],
    )
    def k(batch_hbm, out_hbm, b_v, ub_v, inv_v):
        wid = lax.axis_index("s") * 2 + lax.axis_index("c")

        @pl.when(wid == 0)
        def _():
            pltpu.sync_copy(batch_hbm, b_v)
            lane = lax.broadcasted_iota(jnp.int32, (16,), 0)
            zeros = jnp.zeros((16,), jnp.int32)
            ub_v[pl.ds(0, 16)] = zeros
            for k8 in range(_G // 16):
                g = lane + (16 * k8)
                lo = zeros
                hi = jnp.full((16,), _N, jnp.int32)
                for _ in range(17):
                    active = lo < hi
                    mid = lax.shift_right_arithmetic(lo + hi, 1)
                    v = plsc.load_gather(b_v, [jnp.minimum(mid, _N - 1)])
                    take = jnp.logical_and(active, v <= g)
                    lo = jnp.where(take, mid + 1, lo)
                    hi = jnp.where(jnp.logical_and(active, v > g), mid, hi)
                ub_v[pl.ds(8 + 16 * k8, 16)] = lo
            for k8 in range(_G // 16):
                cur = ub_v[pl.ds(8 + 16 * k8, 16)]
                prev = plsc.load_gather(ub_v, [lane + (7 + 16 * k8)])
                deg = (cur - prev).astype(jnp.float32)
                inv_v[pl.ds(16 * k8, 16)] = _rsqrt_newton(deg)
            pltpu.sync_copy(inv_v, out_hbm)

    return k(batch)


def _scale_body(x_ref, b_ref, inv_ref, o_ref):
    b = b_ref[0, 0, :]
    inv = inv_ref[0, :]
    gid = lax.broadcasted_iota(jnp.int32, (_ROWS, _G), 1)
    eq = b[:, None] == gid
    scale = jnp.sum(jnp.where(eq, inv[None, :], 0.0), axis=1)
    o_ref[:, :] = x_ref[:, :] * scale[:, None]


def kernel(x, batch):
    b32 = batch.astype(jnp.int32)
    inv = _sc_inv_sqrt_deg(b32)
    batch3 = b32.reshape(_N // _ROWS, 1, _ROWS)
    return pl.pallas_call(
        _scale_body,
        grid=(_N // _ROWS,),
        in_specs=[
            pl.BlockSpec((_ROWS, 512), lambda i: (i, 0)),
            pl.BlockSpec((1, 1, _ROWS), lambda i: (i, 0, 0)),
            pl.BlockSpec((1, _G), lambda i: (0, 0)),
        ],
        out_specs=pl.BlockSpec((_ROWS, 512), lambda i: (i, 0)),
        out_shape=jax.ShapeDtypeStruct((_N, 512), jnp.float32),
        compiler_params=pltpu.CompilerParams(
            dimension_semantics=("arbitrary",),
        ),
    )(x, batch3, inv.reshape(1, _G))


# speedup vs baseline: 4.5864x; 1.0036x over previous
import functools

import jax
import jax.numpy as jnp
from jax import lax
from jax.experimental import pallas as pl
from jax.experimental.pallas import tpu as pltpu
from jax.experimental.pallas import tpu_sc as plsc

_N = 100000
_G = 128
_ROWS = 5000


def _rsqrt_newton(x):
    i = plsc.bitcast(x, jnp.int32)
    i = jnp.int32(0x5F3759DF) - lax.shift_right_arithmetic(i, 1)
    y = plsc.bitcast(i, jnp.float32)
    for _ in range(3):
        y = y * (1.5 - 0.5 * x * y * y)
    return y


def _sc_inv_sqrt_deg(batch):
    mesh = plsc.VectorSubcoreMesh(core_axis_name="c", subcore_axis_name="s")

    @functools.partial(
        pl.kernel,
        mesh=mesh,
        compiler_params=pltpu.CompilerParams(needs_layout_passes=False),
        out_type=jax.ShapeDtypeStruct((_G,), jnp.float32),
        scratch_types=[
            pltpu.VMEM((_N,), jnp.int32),
            pltpu.VMEM((_G + 16,), jnp.int32),
            pltpu.VMEM((_G,), jnp.float32),
        ],
    )
    def k(batch_hbm, out_hbm, b_v, ub_v, inv_v):
        wid = lax.axis_index("s") * 2 + lax.axis_index("c")

        @pl.when(wid == 0)
        def _():
            pltpu.sync_copy(batch_hbm, b_v)
            lane = lax.broadcasted_iota(jnp.int32, (16,), 0)
            zeros = jnp.zeros((16,), jnp.int32)
            ub_v[pl.ds(0, 16)] = zeros
            for k8 in range(_G // 16):
                g = lane + (16 * k8)
                lo = zeros
                hi = jnp.full((16,), _N, jnp.int32)
                for _ in range(17):
                    active = lo < hi
                    mid = lax.shift_right_arithmetic(lo + hi, 1)
                    v = plsc.load_gather(b_v, [jnp.minimum(mid, _N - 1)])
                    take = jnp.logical_and(active, v <= g)
                    lo = jnp.where(take, mid + 1, lo)
                    hi = jnp.where(jnp.logical_and(active, v > g), mid, hi)
                ub_v[pl.ds(8 + 16 * k8, 16)] = lo
            for k8 in range(_G // 16):
                cur = ub_v[pl.ds(8 + 16 * k8, 16)]
                prev = plsc.load_gather(ub_v, [lane + (7 + 16 * k8)])
                deg = (cur - prev).astype(jnp.float32)
                inv_v[pl.ds(16 * k8, 16)] = _rsqrt_newton(deg)
            pltpu.sync_copy(inv_v, out_hbm)

    return k(batch)


def _scale_body(x_ref, b_ref, inv_ref, o_ref):
    b = b_ref[0, 0, :]
    inv = inv_ref[0, :]
    gid = lax.broadcasted_iota(jnp.int32, (_ROWS, _G), 1)
    eq = b[:, None] == gid
    scale = jnp.sum(jnp.where(eq, inv[None, :], 0.0), axis=1)
    o_ref[:, :] = x_ref[:, :] * scale[:, None]


def kernel(x, batch):
    b32 = batch.astype(jnp.int32)
    inv = _sc_inv_sqrt_deg(b32)
    batch3 = b32.reshape(_N // _ROWS, 1, _ROWS)
    return pl.pallas_call(
        _scale_body,
        grid=(_N // _ROWS,),
        in_specs=[
            pl.BlockSpec((_ROWS, 512), lambda i: (i, 0)),
            pl.BlockSpec((1, 1, _ROWS), lambda i: (i, 0, 0)),
            pl.BlockSpec((1, _G), lambda i: (0, 0)),
        ],
        out_specs=pl.BlockSpec((_ROWS, 512), lambda i: (i, 0)),
        out_shape=jax.ShapeDtypeStruct((_N, 512), jnp.float32),
        compiler_params=pltpu.CompilerParams(
            dimension_semantics=("arbitrary",),
        ),
    )(x, batch3, inv.reshape(1, _G))


# speedup vs baseline: 4.6777x; 1.0199x over previous
import functools

import jax
import jax.numpy as jnp
from jax import lax
from jax.experimental import pallas as pl
from jax.experimental.pallas import tpu as pltpu
from jax.experimental.pallas import tpu_sc as plsc

_N = 100000
_G = 128
_ROWS = 5000


def _rsqrt_newton(x):
    i = plsc.bitcast(x, jnp.int32)
    i = jnp.int32(0x5F3759DF) - lax.shift_right_arithmetic(i, 1)
    y = plsc.bitcast(i, jnp.float32)
    for _ in range(3):
        y = y * (1.5 - 0.5 * x * y * y)
    return y


_NT = 16
_CHUNK = 6256
_LAST = _N - (_NT - 1) * _CHUNK


def _sc_inv_sqrt_deg(batch):
    mesh = plsc.VectorSubcoreMesh(core_axis_name="c", subcore_axis_name="s")

    @functools.partial(
        pl.kernel,
        mesh=mesh,
        compiler_params=pltpu.CompilerParams(needs_layout_passes=False),
        out_type=jax.ShapeDtypeStruct((1, _G), jnp.float32),
        scratch_types=[
            pltpu.VMEM((_CHUNK,), jnp.int32),
            pltpu.VMEM((_G,), jnp.int32),
            pltpu.VMEM_SHARED((_NT * _G,), jnp.int32),
            pltpu.VMEM((_NT * _G,), jnp.int32),
            pltpu.VMEM((_G + 16,), jnp.int32),
            pltpu.VMEM((_G,), jnp.float32),
        ],
    )
    def k(batch_hbm, out_hbm, b_v, lub_v, shared, uball_v, ub_v, inv_v):
        c = lax.axis_index("c")
        s = lax.axis_index("s")
        lane = lax.broadcasted_iota(jnp.int32, (16,), 0)
        zeros = jnp.zeros((16,), jnp.int32)

        @pl.when(jnp.logical_and(c == 0, s < _NT - 1))
        def _():
            pltpu.sync_copy(batch_hbm.at[pl.ds(s * _CHUNK, _CHUNK)], b_v)

        @pl.when(jnp.logical_and(c == 0, s == _NT - 1))
        def _():
            pltpu.sync_copy(batch_hbm.at[pl.ds((_NT - 1) * _CHUNK, _LAST)],
                            b_v.at[pl.ds(0, _LAST)])

        @pl.when(c == 0)
        def _():
            n = jnp.where(s == _NT - 1, _LAST, _CHUNK)
            for k8 in range(_G // 16):
                g = lane + (16 * k8)
                lo = zeros
                hi = jnp.full((16,), 1, jnp.int32) * n
                for _ in range(13):
                    active = lo < hi
                    mid = lax.shift_right_arithmetic(lo + hi, 1)
                    v = plsc.load_gather(b_v, [jnp.minimum(mid, n - 1)])
                    take = jnp.logical_and(active, v <= g)
                    lo = jnp.where(take, mid + 1, lo)
                    hi = jnp.where(jnp.logical_and(active, v > g), mid, hi)
                lub_v[pl.ds(16 * k8, 16)] = lo
            pltpu.sync_copy(lub_v, shared.at[pl.ds(s * _G, _G)])

        plsc.subcore_barrier()

        @pl.when(jnp.logical_and(c == 0, s == 0))
        def _():
            pltpu.sync_copy(shared, uball_v)
            ub_v[pl.ds(0, 16)] = zeros
            for k8 in range(_G // 16):
                tot = zeros
                for t in range(_NT):
                    tot = tot + uball_v[pl.ds(t * _G + 16 * k8, 16)]
                ub_v[pl.ds(8 + 16 * k8, 16)] = tot
            for k8 in range(_G // 16):
                cur = ub_v[pl.ds(8 + 16 * k8, 16)]
                prev = plsc.load_gather(ub_v, [lane + (7 + 16 * k8)])
                deg = (cur - prev).astype(jnp.float32)
                inv_v[pl.ds(16 * k8, 16)] = _rsqrt_newton(deg)
            pltpu.sync_copy(inv_v, out_hbm.at[0])

    return k(batch)


def _scale_body(x_ref, b_ref, inv_ref, o_ref):
    b = b_ref[0, 0, :]
    inv = inv_ref[0, :]
    gid = lax.broadcasted_iota(jnp.int32, (_ROWS, _G), 1)
    eq = b[:, None] == gid
    scale = jnp.sum(jnp.where(eq, inv[None, :], 0.0), axis=1)
    o_ref[:, :] = x_ref[:, :] * scale[:, None]


def kernel(x, batch):
    b32 = batch.astype(jnp.int32)
    inv = _sc_inv_sqrt_deg(b32)
    batch3 = b32.reshape(_N // _ROWS, 1, _ROWS)
    return pl.pallas_call(
        _scale_body,
        grid=(_N // _ROWS,),
        in_specs=[
            pl.BlockSpec((_ROWS, 512), lambda i: (i, 0)),
            pl.BlockSpec((1, 1, _ROWS), lambda i: (i, 0, 0)),
            pl.BlockSpec((1, _G), lambda i: (0, 0)),
        ],
        out_specs=pl.BlockSpec((_ROWS, 512), lambda i: (i, 0)),
        out_shape=jax.ShapeDtypeStruct((_N, 512), jnp.float32),
        compiler_params=pltpu.CompilerParams(
            dimension_semantics=("arbitrary",),
        ),
    )(x, batch3, inv)


# speedup vs baseline: 4.6989x; 1.0045x over previous
import functools

import jax
import jax.numpy as jnp
from jax import lax
from jax.experimental import pallas as pl
from jax.experimental.pallas import tpu as pltpu
from jax.experimental.pallas import tpu_sc as plsc

_N = 100000
_G = 128
_ROWS = 4096


def _rsqrt_newton(x):
    i = plsc.bitcast(x, jnp.int32)
    i = jnp.int32(0x5F3759DF) - lax.shift_right_arithmetic(i, 1)
    y = plsc.bitcast(i, jnp.float32)
    for _ in range(3):
        y = y * (1.5 - 0.5 * x * y * y)
    return y


_NT = 16
_CHUNK = 6256
_LAST = _N - (_NT - 1) * _CHUNK


def _sc_inv_sqrt_deg(batch):
    mesh = plsc.VectorSubcoreMesh(core_axis_name="c", subcore_axis_name="s")

    @functools.partial(
        pl.kernel,
        mesh=mesh,
        compiler_params=pltpu.CompilerParams(needs_layout_passes=False),
        out_type=jax.ShapeDtypeStruct((1, _G), jnp.float32),
        scratch_types=[
            pltpu.VMEM((_CHUNK,), jnp.int32),
            pltpu.VMEM((_G,), jnp.int32),
            pltpu.VMEM_SHARED((_NT * _G,), jnp.int32),
            pltpu.VMEM((_NT * _G,), jnp.int32),
            pltpu.VMEM((_G + 16,), jnp.int32),
            pltpu.VMEM((_G,), jnp.float32),
        ],
    )
    def k(batch_hbm, out_hbm, b_v, lub_v, shared, uball_v, ub_v, inv_v):
        c = lax.axis_index("c")
        s = lax.axis_index("s")
        lane = lax.broadcasted_iota(jnp.int32, (16,), 0)
        zeros = jnp.zeros((16,), jnp.int32)

        @pl.when(jnp.logical_and(c == 0, s < _NT - 1))
        def _():
            pltpu.sync_copy(batch_hbm.at[pl.ds(s * _CHUNK, _CHUNK)], b_v)

        @pl.when(jnp.logical_and(c == 0, s == _NT - 1))
        def _():
            pltpu.sync_copy(batch_hbm.at[pl.ds((_NT - 1) * _CHUNK, _LAST)],
                            b_v.at[pl.ds(0, _LAST)])

        @pl.when(c == 0)
        def _():
            n = jnp.where(s == _NT - 1, _LAST, _CHUNK)
            for k8 in range(_G // 16):
                g = lane + (16 * k8)
                lo = zeros
                hi = jnp.full((16,), 1, jnp.int32) * n
                for _ in range(13):
                    active = lo < hi
                    mid = lax.shift_right_arithmetic(lo + hi, 1)
                    v = plsc.load_gather(b_v, [jnp.minimum(mid, n - 1)])
                    take = jnp.logical_and(active, v <= g)
                    lo = jnp.where(take, mid + 1, lo)
                    hi = jnp.where(jnp.logical_and(active, v > g), mid, hi)
                lub_v[pl.ds(16 * k8, 16)] = lo
            pltpu.sync_copy(lub_v, shared.at[pl.ds(s * _G, _G)])

        plsc.subcore_barrier()

        @pl.when(jnp.logical_and(c == 0, s == 0))
        def _():
            pltpu.sync_copy(shared, uball_v)
            ub_v[pl.ds(0, 16)] = zeros
            for k8 in range(_G // 16):
                tot = zeros
                for t in range(_NT):
                    tot = tot + uball_v[pl.ds(t * _G + 16 * k8, 16)]
                ub_v[pl.ds(8 + 16 * k8, 16)] = tot
            for k8 in range(_G // 16):
                cur = ub_v[pl.ds(8 + 16 * k8, 16)]
                prev = plsc.load_gather(ub_v, [lane + (7 + 16 * k8)])
                deg = (cur - prev).astype(jnp.float32)
                inv_v[pl.ds(16 * k8, 16)] = _rsqrt_newton(deg)
            pltpu.sync_copy(inv_v, out_hbm.at[0])

    return k(batch)


def _scale_body(x_ref, b_ref, inv_ref, o_ref):
    i = pl.program_id(0)
    b = b_ref[pl.ds(i * _ROWS, _ROWS)]
    inv = inv_ref[0, :]
    gid = lax.broadcasted_iota(jnp.int32, (_ROWS, _G), 1)
    eq = b[:, None] == gid
    scale = jnp.sum(jnp.where(eq, inv[None, :], 0.0), axis=1)
    o_ref[:, :] = x_ref[:, :] * scale[:, None]


def kernel(x, batch):
    b32 = batch.astype(jnp.int32)
    inv = _sc_inv_sqrt_deg(b32)
    grid_n = -(-_N // _ROWS)
    b_pad = jnp.pad(b32, (0, grid_n * _ROWS - _N))
    return pl.pallas_call(
        _scale_body,
        grid=(grid_n,),
        in_specs=[
            pl.BlockSpec((_ROWS, 512), lambda i: (i, 0)),
            pl.BlockSpec((grid_n * _ROWS,), lambda i: (0,)),
            pl.BlockSpec((1, _G), lambda i: (0, 0)),
        ],
        out_specs=pl.BlockSpec((_ROWS, 512), lambda i: (i, 0)),
        out_shape=jax.ShapeDtypeStruct((_N, 512), jnp.float32),
        compiler_params=pltpu.CompilerParams(
            dimension_semantics=("arbitrary",),
        ),
    )(x, b_pad, inv)


# speedup vs baseline: 4.7936x; 1.0202x over previous
import functools

import jax
import jax.numpy as jnp
from jax import lax
from jax.experimental import pallas as pl
from jax.experimental.pallas import tpu as pltpu
from jax.experimental.pallas import tpu_sc as plsc

_N = 100000
_G = 128
_ROWS = 4096

_NT = 32
_CHUNK = 3128
_LAST = _N - (_NT - 1) * _CHUNK


def _sc_partial_counts(batch):
    mesh = plsc.VectorSubcoreMesh(core_axis_name="c", subcore_axis_name="s")

    @functools.partial(
        pl.kernel,
        mesh=mesh,
        compiler_params=pltpu.CompilerParams(needs_layout_passes=False),
        out_type=jax.ShapeDtypeStruct((_NT, _G), jnp.int32),
        scratch_types=[
            pltpu.VMEM((_CHUNK,), jnp.int32),
            pltpu.VMEM((_G,), jnp.int32),
        ],
    )
    def k(batch_hbm, out_hbm, b_v, lub_v):
        c = lax.axis_index("c")
        s = lax.axis_index("s")
        wid = s * 2 + c
        lane = lax.broadcasted_iota(jnp.int32, (16,), 0)

        @pl.when(wid < _NT - 1)
        def _():
            pltpu.sync_copy(batch_hbm.at[pl.ds(wid * _CHUNK, _CHUNK)], b_v)

        @pl.when(wid == _NT - 1)
        def _():
            pltpu.sync_copy(batch_hbm.at[pl.ds((_NT - 1) * _CHUNK, _LAST)],
                            b_v.at[pl.ds(0, _LAST)])

        n = jnp.where(wid == _NT - 1, _LAST, _CHUNK)
        for k8 in range(_G // 16):
            g = lane + (16 * k8)

            def step(_, carry):
                lo, hi = carry
                active = lo < hi
                mid = lax.shift_right_arithmetic(lo + hi, 1)
                v = plsc.load_gather(b_v, [jnp.minimum(mid, n - 1)])
                take = jnp.logical_and(active, v <= g)
                lo = jnp.where(take, mid + 1, lo)
                hi = jnp.where(jnp.logical_and(active, v > g), mid, hi)
                return lo, hi

            lo, _hi = lax.fori_loop(
                0, 12,
                step, (jnp.zeros((16,), jnp.int32), jnp.full((16,), 1, jnp.int32) * n))
            lub_v[pl.ds(16 * k8, 16)] = lo
        pltpu.sync_copy(lub_v, out_hbm.at[wid])

    return k(batch)


def _scale_body(x_ref, b_ref, cnt_ref, o_ref):
    i = pl.program_id(0)
    ub = jnp.sum(cnt_ref[:, :].astype(jnp.float32), axis=0, keepdims=True)
    prev = pltpu.roll(ub, 1, axis=1)
    lane0 = lax.broadcasted_iota(jnp.int32, (1, _G), 1) == 0
    deg = ub - jnp.where(lane0, 0.0, prev)
    inv = lax.rsqrt(deg)[0, :]
    b = b_ref[pl.ds(i * _ROWS, _ROWS)]
    gid = lax.broadcasted_iota(jnp.int32, (_ROWS, _G), 1)
    eq = b[:, None] == gid
    scale = jnp.sum(jnp.where(eq, inv[None, :], 0.0), axis=1)
    o_ref[:, :] = x_ref[:, :] * scale[:, None]


def kernel(x, batch):
    b32 = batch.astype(jnp.int32)
    cnt = _sc_partial_counts(b32)
    grid_n = -(-_N // _ROWS)
    b_pad = jnp.pad(b32, (0, grid_n * _ROWS - _N))
    return pl.pallas_call(
        _scale_body,
        grid=(grid_n,),
        in_specs=[
            pl.BlockSpec((_ROWS, 512), lambda i: (i, 0)),
            pl.BlockSpec((grid_n * _ROWS,), lambda i: (0,)),
            pl.BlockSpec((_NT, _G), lambda i: (0, 0)),
        ],
        out_specs=pl.BlockSpec((_ROWS, 512), lambda i: (i, 0)),
        out_shape=jax.ShapeDtypeStruct((_N, 512), jnp.float32),
        compiler_params=pltpu.CompilerParams(
            dimension_semantics=("arbitrary",),
        ),
    )(x, b_pad, cnt)


# speedup vs baseline: 4.8022x; 1.0018x over previous
import functools

import jax
import jax.numpy as jnp
from jax import lax
from jax.experimental import pallas as pl
from jax.experimental.pallas import tpu as pltpu
from jax.experimental.pallas import tpu_sc as plsc

_N = 100000
_G = 128
_ROWS = 4096

_NT = 32
_CHUNK = 3128
_LAST = _N - (_NT - 1) * _CHUNK


def _sc_partial_counts(batch):
    mesh = plsc.VectorSubcoreMesh(core_axis_name="c", subcore_axis_name="s")

    @functools.partial(
        pl.kernel,
        mesh=mesh,
        compiler_params=pltpu.CompilerParams(needs_layout_passes=False),
        out_type=jax.ShapeDtypeStruct((_NT, _G), jnp.int32),
        scratch_types=[
            pltpu.VMEM((_CHUNK,), jnp.int32),
            pltpu.VMEM((_G,), jnp.int32),
        ],
    )
    def k(batch_hbm, out_hbm, b_v, lub_v):
        c = lax.axis_index("c")
        s = lax.axis_index("s")
        wid = s * 2 + c
        lane = lax.broadcasted_iota(jnp.int32, (16,), 0)

        @pl.when(wid < _NT - 1)
        def _():
            pltpu.sync_copy(batch_hbm.at[pl.ds(wid * _CHUNK, _CHUNK)], b_v)

        @pl.when(wid == _NT - 1)
        def _():
            pltpu.sync_copy(batch_hbm.at[pl.ds((_NT - 1) * _CHUNK, _LAST)],
                            b_v.at[pl.ds(0, _LAST)])

        n = jnp.where(wid == _NT - 1, _LAST, _CHUNK)

        def per_vec(k8, _):
            g = lane + 16 * k8

            def step(_, carry):
                lo, hi = carry
                active = lo < hi
                mid = lax.shift_right_arithmetic(lo + hi, 1)
                v = plsc.load_gather(b_v, [jnp.minimum(mid, n - 1)])
                take = jnp.logical_and(active, v <= g)
                lo = jnp.where(take, mid + 1, lo)
                hi = jnp.where(jnp.logical_and(active, v > g), mid, hi)
                return lo, hi

            lo, _hi = lax.fori_loop(
                0, 12,
                step, (jnp.zeros((16,), jnp.int32), jnp.full((16,), 1, jnp.int32) * n))
            lub_v[pl.ds(k8 * 16, 16)] = lo
            return 0

        lax.fori_loop(0, _G // 16, per_vec, 0)
        pltpu.sync_copy(lub_v, out_hbm.at[wid])

    return k(batch)


def _scale_body(x_ref, b_ref, cnt_ref, o_ref):
    i = pl.program_id(0)
    ub = jnp.sum(cnt_ref[:, :].astype(jnp.float32), axis=0, keepdims=True)
    prev = pltpu.roll(ub, 1, axis=1)
    lane0 = lax.broadcasted_iota(jnp.int32, (1, _G), 1) == 0
    deg = ub - jnp.where(lane0, 0.0, prev)
    inv = lax.rsqrt(deg)[0, :]
    b = b_ref[pl.ds(i * _ROWS, _ROWS)]
    gid = lax.broadcasted_iota(jnp.int32, (_ROWS, _G), 1)
    eq = b[:, None] == gid
    scale = jnp.sum(jnp.where(eq, inv[None, :], 0.0), axis=1)
    o_ref[:, :] = x_ref[:, :] * scale[:, None]


def kernel(x, batch):
    b32 = batch.astype(jnp.int32)
    cnt = _sc_partial_counts(b32)
    grid_n = -(-_N // _ROWS)
    b_pad = jnp.pad(b32, (0, grid_n * _ROWS - _N))
    return pl.pallas_call(
        _scale_body,
        grid=(grid_n,),
        in_specs=[
            pl.BlockSpec((_ROWS, 512), lambda i: (i, 0)),
            pl.BlockSpec((grid_n * _ROWS,), lambda i: (0,)),
            pl.BlockSpec((_NT, _G), lambda i: (0, 0)),
        ],
        out_specs=pl.BlockSpec((_ROWS, 512), lambda i: (i, 0)),
        out_shape=jax.ShapeDtypeStruct((_N, 512), jnp.float32),
        compiler_params=pltpu.CompilerParams(
            dimension_semantics=("arbitrary",),
        ),
    )(x, b_pad, cnt)
